# baseline (speedup 1.0000x reference)
import jax
import jax.numpy as jnp
from jax.experimental import pallas as pl
from jax.experimental.pallas import tpu as pltpu

_N = 256
_D = 4096
_K = 5
_INV_TEMPERATURE = 1.0 / 0.07
_CHUNK = 2048
_HALF = _D // _CHUNK
_STEPS = 2 * _HALF


def _gram(x):
    return jax.lax.dot_general(
        x, x, (((1,), (1,)), ((), ())), preferred_element_type=jnp.float32
    )


def _nncon_loss_kernel(features_ref, feat_t_g_ref, out_ref, sim_acc, adc_acc,
                       mask_ref):
    i = pl.program_id(0)

    @pl.when(i == 0)
    def _sim_init():
        sim_acc[...] = _gram(feat_t_g_ref[...])

    @pl.when((i > 0) & (i < _HALF))
    def _sim_accum():
        sim_acc[...] += _gram(feat_t_g_ref[...])

    @pl.when(i == _HALF)
    def _adc_init():
        adc_acc[...] = _gram(features_ref[...])

    @pl.when(i > _HALF)
    def _adc_accum():
        adc_acc[...] += _gram(features_ref[...])

    @pl.when(i == _HALF)
    def _topk_mask():
        col = jax.lax.broadcasted_iota(jnp.int32, (_N, _N), 1)
        work = sim_acc[...]
        mask = jnp.zeros((_N, _N), dtype=jnp.float32)
        for _ in range(_K):
            row_max = jnp.max(work, axis=1, keepdims=True)
            at_max = work == row_max
            first = jnp.min(jnp.where(at_max, col, _N), axis=1, keepdims=True)
            sel = col == first
            mask = mask + sel.astype(jnp.float32)
            work = jnp.where(sel, -jnp.inf, work)

        row = jax.lax.broadcasted_iota(jnp.int32, (_N, _N), 0)
        mask_ref[...] = mask * (row != col).astype(jnp.float32)

    @pl.when(i == _STEPS - 1)
    def _finish():
        col = jax.lax.broadcasted_iota(jnp.int32, (_N, _N), 1)
        row = jax.lax.broadcasted_iota(jnp.int32, (_N, _N), 0)
        off_diag = (row != col).astype(jnp.float32)
        mask = mask_ref[...]

        adc = adc_acc[...] * _INV_TEMPERATURE
        logits_max = jnp.max(adc, axis=1, keepdims=True)
        logits = adc - logits_max

        exp_sum = jnp.sum(jnp.exp(logits) * off_diag, axis=1, keepdims=True)
        log_prob = logits - jnp.log(exp_sum)

        msum = jnp.sum(mask, axis=1)
        denom = jnp.where(msum == 0.0, 1.0, msum)
        mean_log_prob_pos = jnp.sum(mask * log_prob, axis=1) / denom

        out_ref[...] = (-jnp.sum(mean_log_prob_pos) / _N).reshape(1, 1)


def _g_index(i):
    return (0, jax.lax.min(i, _HALF - 1))


def _f_index(i):
    return (0, jax.lax.max(i - _HALF, 0))


@jax.jit
def kernel(features, feat_t_g):
    out = pl.pallas_call(
        _nncon_loss_kernel,
        grid=(_STEPS,),
        in_specs=[
            pl.BlockSpec((_N, _CHUNK), _f_index),
            pl.BlockSpec((_N, _CHUNK), _g_index),
        ],
        out_specs=pl.BlockSpec((1, 1), lambda i: (0, 0)),
        out_shape=jax.ShapeDtypeStruct((1, 1), jnp.float32),
        scratch_shapes=[
            pltpu.VMEM((_N, _N), jnp.float32),
            pltpu.VMEM((_N, _N), jnp.float32),
            pltpu.VMEM((_N, _N), jnp.float32),
        ],
        compiler_params=pltpu.CompilerParams(
            dimension_semantics=("arbitrary",),
        ),
    )(features, feat_t_g)
    return out[0, 0]


# speedup vs baseline: 1.0092x; 1.0092x over previous
import jax
import jax.numpy as jnp
from jax.experimental import pallas as pl
from jax.experimental.pallas import tpu as pltpu

_N = 256
_D = 4096
_K = 5
_INV_TEMPERATURE = 1.0 / 0.07


def _gram(x):
    return jax.lax.dot_general(
        x, x, (((1,), (1,)), ((), ())), preferred_element_type=jnp.float32
    )


def _nncon_loss_kernel(features_hbm, feat_t_g_hbm, out_ref, f_vmem, g_vmem,
                       f_sem, g_sem):
    g_copy = pltpu.make_async_copy(feat_t_g_hbm, g_vmem, g_sem)
    f_copy = pltpu.make_async_copy(features_hbm, f_vmem, f_sem)
    g_copy.start()
    f_copy.start()

    g_copy.wait()
    sim = _gram(g_vmem[...])

    col = jax.lax.broadcasted_iota(jnp.int32, (_N, _N), 1)
    row = jax.lax.broadcasted_iota(jnp.int32, (_N, _N), 0)

    work = sim
    mask = jnp.zeros((_N, _N), dtype=jnp.float32)
    for _ in range(_K):
        row_max = jnp.max(work, axis=1, keepdims=True)
        at_max = work == row_max
        first = jnp.min(jnp.where(at_max, col, _N), axis=1, keepdims=True)
        sel = col == first
        mask = mask + sel.astype(jnp.float32)
        work = jnp.where(sel, -jnp.inf, work)

    off_diag = (row != col).astype(jnp.float32)
    mask = mask * off_diag

    f_copy.wait()
    adc = _gram(f_vmem[...]) * _INV_TEMPERATURE
    logits_max = jnp.max(adc, axis=1, keepdims=True)
    logits = adc - logits_max

    exp_sum = jnp.sum(jnp.exp(logits) * off_diag, axis=1, keepdims=True)
    log_es = jnp.log(exp_sum)[:, 0]

    msum = jnp.sum(mask, axis=1)
    denom = jnp.where(msum == 0.0, 1.0, msum)
    s1 = jnp.sum(mask * logits, axis=1)
    mean_log_prob_pos = (s1 - log_es * msum) / denom

    out_ref[...] = (-jnp.sum(mean_log_prob_pos) / _N).reshape(1, 1)


@jax.jit
def kernel(features, feat_t_g):
    out = pl.pallas_call(
        _nncon_loss_kernel,
        in_specs=[
            pl.BlockSpec(memory_space=pl.ANY),
            pl.BlockSpec(memory_space=pl.ANY),
        ],
        out_specs=pl.BlockSpec(memory_space=pltpu.VMEM),
        out_shape=jax.ShapeDtypeStruct((1, 1), jnp.float32),
        scratch_shapes=[
            pltpu.VMEM((_N, _D), jnp.float32),
            pltpu.VMEM((_N, _D), jnp.float32),
            pltpu.SemaphoreType.DMA,
            pltpu.SemaphoreType.DMA,
        ],
    )(features, feat_t_g)
    return out[0, 0]
